# Initial kernel scaffold; baseline (speedup 1.0000x reference)
import jax
import jax.numpy as jnp
from jax.experimental import pallas as pl


def kernel(observe, cidx, pidx, K, C, P):
    raise NotImplementedError("write your pallas kernel here")


# speedup vs baseline: 5.0132x; 5.0132x over previous
import jax
import jax.numpy as jnp
from jax import lax
from jax.experimental import pallas as pl
from jax.experimental.pallas import tpu as pltpu
from jax.experimental.pallas import tpu_sc as plsc

_LANES = 16
_NW = 32
_CH = 3200
_IB = 128
_KSUB = _CH // _IB


def _reproj_body(obs_hbm, cidx_hbm, pidx3_hbm, kc_hbm, px_hbm, py_hbm,
                 pz_hbm, out_hbm, kc_v, cidx_v, pidx_v, obs_v, px_v, py_v,
                 pz_v, out_v, sem):
    n_obs = cidx_hbm.shape[0]
    nchunks = n_obs // _CH
    cpt = (nchunks + _NW - 1) // _NW
    wid = lax.axis_index("s") * 2 + lax.axis_index("c")

    pltpu.sync_copy(kc_hbm, kc_v)

    iota = lax.broadcasted_iota(jnp.int32, (_LANES,), 0)

    @pl.loop(0, cpt)
    def _chunk(t):
        chunk = wid + t * _NW

        @pl.when(chunk < nchunks)
        def _():
            base = chunk * _CH
            pltpu.sync_copy(cidx_hbm.at[pl.ds(base, _CH)], cidx_v)
            pltpu.sync_copy(pidx3_hbm.at[chunk], pidx_v)
            pltpu.sync_copy(obs_hbm.at[pl.ds(2 * base, 2 * _CH)], obs_v)

            @pl.loop(0, _KSUB)
            def _fire(j):
                idx = pidx_v.at[j]
                dst = pl.ds(j * _IB, _IB)
                pltpu.make_async_copy(px_hbm.at[idx], px_v.at[dst], sem).start()
                pltpu.make_async_copy(py_hbm.at[idx], py_v.at[dst], sem).start()
                pltpu.make_async_copy(pz_hbm.at[idx], pz_v.at[dst], sem).start()

            @pl.loop(0, _KSUB)
            def _drain(j):
                idx = pidx_v.at[j]
                dst = pl.ds(j * _IB, _IB)
                pltpu.make_async_copy(px_hbm.at[idx], px_v.at[dst], sem).wait()
                pltpu.make_async_copy(py_hbm.at[idx], py_v.at[dst], sem).wait()
                pltpu.make_async_copy(pz_hbm.at[idx], pz_v.at[dst], sem).wait()

            @pl.loop(0, _CH // _LANES)
            def _compute(i):
                off = i * _LANES
                sl = pl.ds(off, _LANES)
                cam10 = cidx_v[sl] * 10

                def kc(j):
                    return plsc.load_gather(kc_v, [cam10 + j])

                f = kc(0)
                k1 = kc(1)
                k2 = kc(2)
                tx = kc(3)
                ty = kc(4)
                tz = kc(5)
                qx = kc(6)
                qy = kc(7)
                qz = kc(8)
                qw = kc(9)

                px = px_v[sl]
                py = py_v[sl]
                pz = pz_v[sl]

                t1x = qy * pz - qz * py + qw * px
                t1y = qz * px - qx * pz + qw * py
                t1z = qx * py - qy * px + qw * pz
                cx = px + 2.0 * (qy * t1z - qz * t1y) + tx
                cy = py + 2.0 * (qz * t1x - qx * t1z) + ty
                cz = pz + 2.0 * (qx * t1y - qy * t1x) + tz

                inv = -1.0 / cz
                nx = cx * inv
                ny = cy * inv
                r = nx * nx + ny * ny
                fd = f * (1.0 + k1 * r + k2 * (r * r))

                flat = (iota + off) * 2
                ox = plsc.load_gather(obs_v, [flat])
                oy = plsc.load_gather(obs_v, [flat + 1])
                plsc.store_scatter(out_v, [flat], fd * nx - ox)
                plsc.store_scatter(out_v, [flat + 1], fd * ny - oy)

            pltpu.sync_copy(out_v, out_hbm.at[pl.ds(2 * base, 2 * _CH)])


def kernel(observe, cidx, pidx, K, C, P):
    n_obs = observe.shape[0]
    n_cams = K.shape[0]
    KC = jnp.concatenate([K, C], axis=1).reshape(-1)
    cidx = cidx.astype(jnp.int32)
    pidx3 = pidx.astype(jnp.int32).reshape(-1, _KSUB, _IB)
    px, py, pz = P[:, 0], P[:, 1], P[:, 2]
    obs_flat = observe.reshape(-1)

    mesh = plsc.VectorSubcoreMesh(core_axis_name="c", subcore_axis_name="s")
    kfun = pl.kernel(
        _reproj_body,
        out_type=jax.ShapeDtypeStruct((2 * n_obs,), jnp.float32),
        mesh=mesh,
        compiler_params=pltpu.CompilerParams(needs_layout_passes=False),
        scratch_types=[
            pltpu.VMEM((n_cams * 10,), jnp.float32),
            pltpu.VMEM((_CH,), jnp.int32),
            pltpu.VMEM((_KSUB, _IB), jnp.int32),
            pltpu.VMEM((2 * _CH,), jnp.float32),
            pltpu.VMEM((_CH,), jnp.float32),
            pltpu.VMEM((_CH,), jnp.float32),
            pltpu.VMEM((_CH,), jnp.float32),
            pltpu.VMEM((2 * _CH,), jnp.float32),
            pltpu.SemaphoreType.DMA,
        ],
    )
    out = kfun(obs_flat, cidx, pidx3, KC, px, py, pz)
    return out.reshape(n_obs, 2)


# speedup vs baseline: 33.9318x; 6.7685x over previous
import jax
import jax.numpy as jnp
from jax import lax
from jax.experimental import pallas as pl
from jax.experimental.pallas import tpu as pltpu
from jax.experimental.pallas import tpu_sc as plsc

_LANES = 16
_NW = 32
_CH = 3200
_IB = 128
_KSUB = _CH // _IB


def _reproj_body(obsx_hbm, obsy_hbm, cidx_hbm, pidx_hbm, kc_hbm, px_hbm,
                 py_hbm, pz_hbm, resx_hbm, resy_hbm, kc_v, cidx_v, pidx_v,
                 obsx_v, obsy_v, px_v, py_v, pz_v, resx_v, resy_v, sem):
    n_obs = cidx_hbm.shape[0]
    nchunks = n_obs // _CH
    cpt = (nchunks + _NW - 1) // _NW
    wid = lax.axis_index("s") * 2 + lax.axis_index("c")

    pltpu.sync_copy(kc_hbm, kc_v)

    @pl.loop(0, cpt)
    def _chunk(t):
        chunk = wid + t * _NW

        @pl.when(chunk < nchunks)
        def _():
            base = chunk * _CH
            sl_hbm = pl.ds(base, _CH)
            pltpu.sync_copy(pidx_hbm.at[sl_hbm], pidx_v)
            pltpu.sync_copy(cidx_hbm.at[sl_hbm], cidx_v)
            pltpu.sync_copy(obsx_hbm.at[sl_hbm], obsx_v)
            pltpu.sync_copy(obsy_hbm.at[sl_hbm], obsy_v)

            @pl.loop(0, _KSUB)
            def _fire(j):
                blk = pl.ds(j * _IB, _IB)
                idx = pidx_v.at[blk]
                pltpu.make_async_copy(px_hbm.at[idx], px_v.at[blk], sem).start()
                pltpu.make_async_copy(py_hbm.at[idx], py_v.at[blk], sem).start()
                pltpu.make_async_copy(pz_hbm.at[idx], pz_v.at[blk], sem).start()

            @pl.loop(0, _KSUB)
            def _drain(j):
                blk = pl.ds(j * _IB, _IB)
                idx = pidx_v.at[blk]
                pltpu.make_async_copy(px_hbm.at[idx], px_v.at[blk], sem).wait()
                pltpu.make_async_copy(py_hbm.at[idx], py_v.at[blk], sem).wait()
                pltpu.make_async_copy(pz_hbm.at[idx], pz_v.at[blk], sem).wait()

            @pl.loop(0, _CH // _LANES)
            def _compute(i):
                sl = pl.ds(i * _LANES, _LANES)
                cam10 = cidx_v[sl] * 10

                def kc(j):
                    return plsc.load_gather(kc_v, [cam10 + j])

                f = kc(0)
                k1 = kc(1)
                k2 = kc(2)
                tx = kc(3)
                ty = kc(4)
                tz = kc(5)
                qx = kc(6)
                qy = kc(7)
                qz = kc(8)
                qw = kc(9)

                px = px_v[sl]
                py = py_v[sl]
                pz = pz_v[sl]

                t1x = qy * pz - qz * py + qw * px
                t1y = qz * px - qx * pz + qw * py
                t1z = qx * py - qy * px + qw * pz
                cx = px + 2.0 * (qy * t1z - qz * t1y) + tx
                cy = py + 2.0 * (qz * t1x - qx * t1z) + ty
                cz = pz + 2.0 * (qx * t1y - qy * t1x) + tz

                inv = -1.0 / cz
                nx = cx * inv
                ny = cy * inv
                r = nx * nx + ny * ny
                fd = f * (1.0 + k1 * r + k2 * (r * r))

                resx_v[sl] = fd * nx - obsx_v[sl]
                resy_v[sl] = fd * ny - obsy_v[sl]

            pltpu.sync_copy(resx_v, resx_hbm.at[sl_hbm])
            pltpu.sync_copy(resy_v, resy_hbm.at[sl_hbm])


def kernel(observe, cidx, pidx, K, C, P):
    n_obs = observe.shape[0]
    KC = jnp.concatenate([K, C], axis=1).reshape(-1)
    cidx = cidx.astype(jnp.int32)
    pidx = pidx.astype(jnp.int32)
    obsx, obsy = observe[:, 0], observe[:, 1]
    px, py, pz = P[:, 0], P[:, 1], P[:, 2]

    mesh = plsc.VectorSubcoreMesh(core_axis_name="c", subcore_axis_name="s")
    kfun = pl.kernel(
        _reproj_body,
        out_type=(
            jax.ShapeDtypeStruct((n_obs,), jnp.float32),
            jax.ShapeDtypeStruct((n_obs,), jnp.float32),
        ),
        mesh=mesh,
        compiler_params=pltpu.CompilerParams(needs_layout_passes=False),
        scratch_types=[
            pltpu.VMEM((KC.shape[0],), jnp.float32),
            pltpu.VMEM((_CH,), jnp.int32),
            pltpu.VMEM((_CH,), jnp.int32),
            pltpu.VMEM((_CH,), jnp.float32),
            pltpu.VMEM((_CH,), jnp.float32),
            pltpu.VMEM((_CH,), jnp.float32),
            pltpu.VMEM((_CH,), jnp.float32),
            pltpu.VMEM((_CH,), jnp.float32),
            pltpu.VMEM((_CH,), jnp.float32),
            pltpu.VMEM((_CH,), jnp.float32),
            pltpu.SemaphoreType.DMA,
        ],
    )
    resx, resy = kfun(obsx, obsy, cidx, pidx, KC, px, py, pz)
    return jnp.stack([resx, resy], axis=-1)


# speedup vs baseline: 52.1493x; 1.5369x over previous
import jax
import jax.numpy as jnp
from jax import lax
from jax.experimental import pallas as pl
from jax.experimental.pallas import tpu as pltpu
from jax.experimental.pallas import tpu_sc as plsc

_LANES = 16
_NW = 32
_CH = 3200
_IB = 128
_KSUB = _CH // _IB


def _reproj_body(cidx_hbm, pidx_hbm, kct_hbm, px_hbm, py_hbm, pz_hbm,
                 projx_hbm, projy_hbm, kct_v, cidx_v, pidx_v, px_v, py_v,
                 pz_v, resx_v, resy_v, sem_lin, sem_pa, sem_pb):
    n_obs = cidx_hbm.shape[0]
    n_cams = kct_hbm.shape[0] // 10
    nchunks = n_obs // _CH
    cpt = (nchunks + _NW - 1) // _NW
    wid = lax.axis_index("s") * 2 + lax.axis_index("c")

    pltpu.sync_copy(kct_hbm, kct_v)

    def lin_copies(chunk, slot):
        src = pl.ds(chunk * _CH, _CH)
        dst = pl.ds(slot * _CH, _CH)
        return (
            pltpu.make_async_copy(cidx_hbm.at[src], cidx_v.at[dst], sem_lin),
            pltpu.make_async_copy(pidx_hbm.at[src], pidx_v.at[dst], sem_lin),
        )

    def load_linear(chunk, slot):
        for c in lin_copies(chunk, slot):
            c.start()
        for c in lin_copies(chunk, slot):
            c.wait()

    def p_copies(j, slot, sem):
        blk = pl.ds(slot * _CH + j * _IB, _IB)
        idx = pidx_v.at[blk]
        return (
            pltpu.make_async_copy(px_hbm.at[idx], px_v.at[blk], sem),
            pltpu.make_async_copy(py_hbm.at[idx], py_v.at[blk], sem),
            pltpu.make_async_copy(pz_hbm.at[idx], pz_v.at[blk], sem),
        )

    def fire_p(slot, sem):
        @pl.loop(0, _KSUB)
        def _(j):
            for c in p_copies(j, slot, sem):
                c.start()

    def drain_p(slot, sem):
        @pl.loop(0, _KSUB)
        def _(j):
            for c in p_copies(j, slot, sem):
                c.wait()

    def compute_store(chunk, slot):
        soff = slot * _CH

        @pl.loop(0, _CH // _LANES)
        def _compute(i):
            sl = pl.ds(soff + i * _LANES, _LANES)
            osl = pl.ds(i * _LANES, _LANES)
            cam = cidx_v[sl]

            def kc(j):
                return plsc.load_gather(
                    kct_v.at[pl.ds(j * n_cams, n_cams)], [cam])

            f = kc(0)
            k1 = kc(1)
            k2 = kc(2)
            tx = kc(3)
            ty = kc(4)
            tz = kc(5)
            qx = kc(6)
            qy = kc(7)
            qz = kc(8)
            qw = kc(9)

            px = px_v[sl]
            py = py_v[sl]
            pz = pz_v[sl]

            t1x = qy * pz - qz * py + qw * px
            t1y = qz * px - qx * pz + qw * py
            t1z = qx * py - qy * px + qw * pz
            cx = px + 2.0 * (qy * t1z - qz * t1y) + tx
            cy = py + 2.0 * (qz * t1x - qx * t1z) + ty
            cz = pz + 2.0 * (qx * t1y - qy * t1x) + tz

            inv = -1.0 / cz
            nx = cx * inv
            ny = cy * inv
            r = nx * nx + ny * ny
            fd = f * (1.0 + k1 * r + k2 * (r * r))

            resx_v[osl] = fd * nx
            resy_v[osl] = fd * ny

        out_sl = pl.ds(chunk * _CH, _CH)
        pltpu.sync_copy(resx_v, projx_hbm.at[out_sl])
        pltpu.sync_copy(resy_v, projy_hbm.at[out_sl])

    def stage(t, slot, sem_this, sem_next):
        chunk = wid + t * _NW
        nxt = chunk + _NW

        @pl.when(nxt < nchunks)
        def _():
            load_linear(nxt, 1 - slot)
            fire_p(1 - slot, sem_next)

        @pl.when(chunk < nchunks)
        def _():
            drain_p(slot, sem_this)
            compute_store(chunk, slot)

    load_linear(wid, 0)
    fire_p(0, sem_pa)

    @pl.loop(0, cpt // 2)
    def _pair(u):
        stage(2 * u, 0, sem_pa, sem_pb)
        stage(2 * u + 1, 1, sem_pb, sem_pa)


def kernel(observe, cidx, pidx, K, C, P):
    n_obs = observe.shape[0]
    cidx = cidx.astype(jnp.int32)
    pidx = pidx.astype(jnp.int32)
    KCt = jnp.concatenate([K, C], axis=1).T.reshape(-1)
    px, py, pz = P[:, 0], P[:, 1], P[:, 2]

    nchunks = n_obs // _CH
    assert nchunks * _CH == n_obs and ((nchunks + _NW - 1) // _NW) % 2 == 0

    mesh = plsc.VectorSubcoreMesh(core_axis_name="c", subcore_axis_name="s")
    kfun = pl.kernel(
        _reproj_body,
        out_type=(
            jax.ShapeDtypeStruct((n_obs,), jnp.float32),
            jax.ShapeDtypeStruct((n_obs,), jnp.float32),
        ),
        mesh=mesh,
        compiler_params=pltpu.CompilerParams(needs_layout_passes=False),
        scratch_types=[
            pltpu.VMEM((KCt.shape[0],), jnp.float32),
            pltpu.VMEM((2 * _CH,), jnp.int32),
            pltpu.VMEM((2 * _CH,), jnp.int32),
            pltpu.VMEM((2 * _CH,), jnp.float32),
            pltpu.VMEM((2 * _CH,), jnp.float32),
            pltpu.VMEM((2 * _CH,), jnp.float32),
            pltpu.VMEM((_CH,), jnp.float32),
            pltpu.VMEM((_CH,), jnp.float32),
            pltpu.SemaphoreType.DMA,
            pltpu.SemaphoreType.DMA,
            pltpu.SemaphoreType.DMA,
        ],
    )
    projx, projy = kfun(cidx, pidx, KCt, px, py, pz)
    return jnp.stack([projx, projy], axis=-1) - observe


# speedup vs baseline: 58.8036x; 1.1276x over previous
import jax
import jax.numpy as jnp
from jax import lax
from jax.experimental import pallas as pl
from jax.experimental.pallas import tpu as pltpu
from jax.experimental.pallas import tpu_sc as plsc

_LANES = 16
_NW = 32
_CH = 3200
_IB = 3200
_KSUB = _CH // _IB


def _reproj_body(cidx_hbm, pidx_hbm, kct_hbm, px_hbm, py_hbm, pz_hbm,
                 projx_hbm, projy_hbm, kct_v, cidx_v, pidx_v, px_v, py_v,
                 pz_v, resx_v, resy_v, sem_lin, sem_pa, sem_pb):
    n_obs = cidx_hbm.shape[0]
    n_cams = kct_hbm.shape[0] // 10
    nchunks = n_obs // _CH
    cpt = (nchunks + _NW - 1) // _NW
    wid = lax.axis_index("s") * 2 + lax.axis_index("c")

    pltpu.sync_copy(kct_hbm, kct_v)

    def lin_copies(chunk, slot):
        src = pl.ds(chunk * _CH, _CH)
        dst = pl.ds(slot * _CH, _CH)
        return (
            pltpu.make_async_copy(cidx_hbm.at[src], cidx_v.at[dst], sem_lin),
            pltpu.make_async_copy(pidx_hbm.at[src], pidx_v.at[dst], sem_lin),
        )

    def load_linear(chunk, slot):
        for c in lin_copies(chunk, slot):
            c.start()
        for c in lin_copies(chunk, slot):
            c.wait()

    def p_copies(j, slot, sem):
        blk = pl.ds(slot * _CH + j * _IB, _IB)
        idx = pidx_v.at[blk]
        return (
            pltpu.make_async_copy(px_hbm.at[idx], px_v.at[blk], sem),
            pltpu.make_async_copy(py_hbm.at[idx], py_v.at[blk], sem),
            pltpu.make_async_copy(pz_hbm.at[idx], pz_v.at[blk], sem),
        )

    def fire_p(slot, sem):
        @pl.loop(0, _KSUB)
        def _(j):
            for c in p_copies(j, slot, sem):
                c.start()

    def drain_p(slot, sem):
        @pl.loop(0, _KSUB)
        def _(j):
            for c in p_copies(j, slot, sem):
                c.wait()

    def compute_store(chunk, slot):
        soff = slot * _CH

        @pl.loop(0, _CH // _LANES)
        def _compute(i):
            sl = pl.ds(soff + i * _LANES, _LANES)
            osl = pl.ds(i * _LANES, _LANES)
            cam = cidx_v[sl]

            def kc(j):
                return plsc.load_gather(
                    kct_v.at[pl.ds(j * n_cams, n_cams)], [cam])

            f = kc(0)
            k1 = kc(1)
            k2 = kc(2)
            tx = kc(3)
            ty = kc(4)
            tz = kc(5)
            qx = kc(6)
            qy = kc(7)
            qz = kc(8)
            qw = kc(9)

            px = px_v[sl]
            py = py_v[sl]
            pz = pz_v[sl]

            t1x = qy * pz - qz * py + qw * px
            t1y = qz * px - qx * pz + qw * py
            t1z = qx * py - qy * px + qw * pz
            cx = px + 2.0 * (qy * t1z - qz * t1y) + tx
            cy = py + 2.0 * (qz * t1x - qx * t1z) + ty
            cz = pz + 2.0 * (qx * t1y - qy * t1x) + tz

            inv = -1.0 / cz
            nx = cx * inv
            ny = cy * inv
            r = nx * nx + ny * ny
            fd = f * (1.0 + k1 * r + k2 * (r * r))

            resx_v[osl] = fd * nx
            resy_v[osl] = fd * ny

        out_sl = pl.ds(chunk * _CH, _CH)
        pltpu.sync_copy(resx_v, projx_hbm.at[out_sl])
        pltpu.sync_copy(resy_v, projy_hbm.at[out_sl])

    def stage(t, slot, sem_this, sem_next):
        chunk = wid + t * _NW
        nxt = chunk + _NW

        @pl.when(nxt < nchunks)
        def _():
            load_linear(nxt, 1 - slot)
            fire_p(1 - slot, sem_next)

        @pl.when(chunk < nchunks)
        def _():
            drain_p(slot, sem_this)
            compute_store(chunk, slot)

    load_linear(wid, 0)
    fire_p(0, sem_pa)

    @pl.loop(0, cpt // 2)
    def _pair(u):
        stage(2 * u, 0, sem_pa, sem_pb)
        stage(2 * u + 1, 1, sem_pb, sem_pa)


def kernel(observe, cidx, pidx, K, C, P):
    n_obs = observe.shape[0]
    cidx = cidx.astype(jnp.int32)
    pidx = pidx.astype(jnp.int32)
    KCt = jnp.concatenate([K, C], axis=1).T.reshape(-1)
    px, py, pz = P[:, 0], P[:, 1], P[:, 2]

    nchunks = n_obs // _CH
    assert nchunks * _CH == n_obs and ((nchunks + _NW - 1) // _NW) % 2 == 0

    mesh = plsc.VectorSubcoreMesh(core_axis_name="c", subcore_axis_name="s")
    kfun = pl.kernel(
        _reproj_body,
        out_type=(
            jax.ShapeDtypeStruct((n_obs,), jnp.float32),
            jax.ShapeDtypeStruct((n_obs,), jnp.float32),
        ),
        mesh=mesh,
        compiler_params=pltpu.CompilerParams(needs_layout_passes=False),
        scratch_types=[
            pltpu.VMEM((KCt.shape[0],), jnp.float32),
            pltpu.VMEM((2 * _CH,), jnp.int32),
            pltpu.VMEM((2 * _CH,), jnp.int32),
            pltpu.VMEM((2 * _CH,), jnp.float32),
            pltpu.VMEM((2 * _CH,), jnp.float32),
            pltpu.VMEM((2 * _CH,), jnp.float32),
            pltpu.VMEM((_CH,), jnp.float32),
            pltpu.VMEM((_CH,), jnp.float32),
            pltpu.SemaphoreType.DMA,
            pltpu.SemaphoreType.DMA,
            pltpu.SemaphoreType.DMA,
        ],
    )
    projx, projy = kfun(cidx, pidx, KCt, px, py, pz)
    return jnp.stack([projx, projy], axis=-1) - observe


# speedup vs baseline: 60.0562x; 1.0213x over previous
import jax
import jax.numpy as jnp
from jax import lax
from jax.experimental import pallas as pl
from jax.experimental.pallas import tpu as pltpu
from jax.experimental.pallas import tpu_sc as plsc

_LANES = 16
_NW = 32
_CH = 3200
_IB = 3200
_KSUB = _CH // _IB


def _reproj_body(cidx_hbm, pidx_hbm, kct_hbm, px_hbm, py_hbm, pz_hbm,
                 projx_hbm, projy_hbm, kct_v, cidx_v, pidx_v, px_v, py_v,
                 pz_v, resx_v, resy_v, sem_lin, sem_pa, sem_pb):
    n_obs = cidx_hbm.shape[0]
    n_cams = kct_hbm.shape[0] // 10
    nchunks = n_obs // _CH
    cpt = (nchunks + _NW - 1) // _NW
    wid = lax.axis_index("s") * 2 + lax.axis_index("c")

    pltpu.sync_copy(kct_hbm, kct_v)

    def lin_copies(chunk, slot):
        src = pl.ds(chunk * _CH, _CH)
        dst = pl.ds(slot * _CH, _CH)
        return (
            pltpu.make_async_copy(cidx_hbm.at[src], cidx_v.at[dst], sem_lin),
            pltpu.make_async_copy(pidx_hbm.at[src], pidx_v.at[dst], sem_lin),
        )

    def load_linear(chunk, slot):
        for c in lin_copies(chunk, slot):
            c.start()
        for c in lin_copies(chunk, slot):
            c.wait()

    def p_copies(j, slot, sem):
        blk = pl.ds(slot * _CH + j * _IB, _IB)
        idx = pidx_v.at[blk]
        return (
            pltpu.make_async_copy(px_hbm.at[idx], px_v.at[blk], sem),
            pltpu.make_async_copy(py_hbm.at[idx], py_v.at[blk], sem),
            pltpu.make_async_copy(pz_hbm.at[idx], pz_v.at[blk], sem),
        )

    def fire_p(slot, sem):
        @pl.loop(0, _KSUB)
        def _(j):
            for c in p_copies(j, slot, sem):
                c.start()

    def drain_p(slot, sem):
        @pl.loop(0, _KSUB)
        def _(j):
            for c in p_copies(j, slot, sem):
                c.wait()

    def compute_store(chunk, slot):
        soff = slot * _CH

        @pl.loop(0, _CH // _LANES, unroll=4)
        def _compute(i):
            sl = pl.ds(soff + i * _LANES, _LANES)
            osl = pl.ds(i * _LANES, _LANES)
            cam = cidx_v[sl]

            def kc(j):
                return plsc.load_gather(
                    kct_v.at[pl.ds(j * n_cams, n_cams)], [cam])

            f = kc(0)
            k1 = kc(1)
            k2 = kc(2)
            tx = kc(3)
            ty = kc(4)
            tz = kc(5)
            qx = kc(6)
            qy = kc(7)
            qz = kc(8)
            qw = kc(9)

            px = px_v[sl]
            py = py_v[sl]
            pz = pz_v[sl]

            t1x = qy * pz - qz * py + qw * px
            t1y = qz * px - qx * pz + qw * py
            t1z = qx * py - qy * px + qw * pz
            cx = px + 2.0 * (qy * t1z - qz * t1y) + tx
            cy = py + 2.0 * (qz * t1x - qx * t1z) + ty
            cz = pz + 2.0 * (qx * t1y - qy * t1x) + tz

            inv = -1.0 / cz
            nx = cx * inv
            ny = cy * inv
            r = nx * nx + ny * ny
            fd = f * (1.0 + k1 * r + k2 * (r * r))

            resx_v[osl] = fd * nx
            resy_v[osl] = fd * ny

        out_sl = pl.ds(chunk * _CH, _CH)
        pltpu.sync_copy(resx_v, projx_hbm.at[out_sl])
        pltpu.sync_copy(resy_v, projy_hbm.at[out_sl])

    def stage(t, slot, sem_this, sem_next):
        chunk = wid + t * _NW
        nxt = chunk + _NW

        @pl.when(nxt < nchunks)
        def _():
            load_linear(nxt, 1 - slot)
            fire_p(1 - slot, sem_next)

        @pl.when(chunk < nchunks)
        def _():
            drain_p(slot, sem_this)
            compute_store(chunk, slot)

    load_linear(wid, 0)
    fire_p(0, sem_pa)

    @pl.loop(0, cpt // 2)
    def _pair(u):
        stage(2 * u, 0, sem_pa, sem_pb)
        stage(2 * u + 1, 1, sem_pb, sem_pa)


def kernel(observe, cidx, pidx, K, C, P):
    n_obs = observe.shape[0]
    cidx = cidx.astype(jnp.int32)
    pidx = pidx.astype(jnp.int32)
    KCt = jnp.concatenate([K, C], axis=1).T.reshape(-1)
    px, py, pz = P[:, 0], P[:, 1], P[:, 2]

    nchunks = n_obs // _CH
    assert nchunks * _CH == n_obs and ((nchunks + _NW - 1) // _NW) % 2 == 0

    mesh = plsc.VectorSubcoreMesh(core_axis_name="c", subcore_axis_name="s")
    kfun = pl.kernel(
        _reproj_body,
        out_type=(
            jax.ShapeDtypeStruct((n_obs,), jnp.float32),
            jax.ShapeDtypeStruct((n_obs,), jnp.float32),
        ),
        mesh=mesh,
        compiler_params=pltpu.CompilerParams(needs_layout_passes=False),
        scratch_types=[
            pltpu.VMEM((KCt.shape[0],), jnp.float32),
            pltpu.VMEM((2 * _CH,), jnp.int32),
            pltpu.VMEM((2 * _CH,), jnp.int32),
            pltpu.VMEM((2 * _CH,), jnp.float32),
            pltpu.VMEM((2 * _CH,), jnp.float32),
            pltpu.VMEM((2 * _CH,), jnp.float32),
            pltpu.VMEM((_CH,), jnp.float32),
            pltpu.VMEM((_CH,), jnp.float32),
            pltpu.SemaphoreType.DMA,
            pltpu.SemaphoreType.DMA,
            pltpu.SemaphoreType.DMA,
        ],
    )
    projx, projy = kfun(cidx, pidx, KCt, px, py, pz)
    return jnp.stack([projx, projy], axis=-1) - observe
